# baseline (speedup 1.0000x reference)
import functools

import jax
import jax.numpy as jnp
from jax import lax
from jax.experimental import pallas as pl
from jax.experimental.pallas import tpu as pltpu
from jax.experimental.pallas import tpu_sc as plsc

K_TOP = 2048
TEMPERATURE = 0.1

N = 4194304
NUM_CORES = 2
NUM_SUBCORES = 16
NUM_WORKERS = NUM_CORES * NUM_SUBCORES
LANES = 16
SHARD = N // NUM_WORKERS
CHUNK = 16384
NCHUNK = SHARD // CHUNK
BINS = 65536
UNROLL = 8
MIN_I32 = -(2**31)


def _make_hist_kernel(prefix_pass):
    mesh = plsc.VectorSubcoreMesh(core_axis_name="c", subcore_axis_name="s")

    @functools.partial(
        pl.kernel,
        mesh=mesh,
        out_type=jax.ShapeDtypeStruct((NUM_WORKERS, BINS), jnp.int32),
        compiler_params=pltpu.CompilerParams(
            needs_layout_passes=False, use_tc_tiling_on_sc=True),
        scratch_types=[
            pltpu.VMEM((BINS,), jnp.int32),
            pltpu.VMEM((CHUNK,), jnp.float32),
            pltpu.VMEM((CHUNK,), jnp.float32),
            pltpu.VMEM((LANES,), jnp.int32),
            pltpu.SemaphoreType.DMA,
            pltpu.SemaphoreType.DMA,
        ],
    )
    def hist_kernel(logits_hbm, prefix_hbm, out_hbm, hist_v, buf0, buf1,
                    pref_v, sem0, sem1):
        wid = lax.axis_index("s") * NUM_CORES + lax.axis_index("c")
        base = wid * SHARD
        ones = jnp.ones((LANES,), jnp.int32)
        zeros = jnp.zeros((LANES,), jnp.int32)
        bufs = (buf0, buf1)
        sems = (sem0, sem1)

        pltpu.sync_copy(prefix_hbm, pref_v)
        pvec = pref_v[...]

        copies = [None] * NCHUNK
        copies[0] = pltpu.async_copy(
            logits_hbm.at[pl.ds(base, CHUNK)], buf0, sem0)

        @plsc.parallel_loop(0, BINS // LANES, 1, unroll=UNROLL)
        def _(j):
            hist_v[pl.ds(j * LANES, LANES)] = zeros

        for c in range(NCHUNK):
            if c + 1 < NCHUNK:
                copies[c + 1] = pltpu.async_copy(
                    logits_hbm.at[pl.ds(base + (c + 1) * CHUNK, CHUNK)],
                    bufs[(c + 1) % 2], sems[(c + 1) % 2])
            copies[c].wait()
            buf = bufs[c % 2]

            @plsc.parallel_loop(0, CHUNK // LANES, 1, unroll=UNROLL)
            def _(i):
                x = buf[pl.ds(i * LANES, LANES)]
                b = lax.bitcast_convert_type(x, jnp.int32)
                if not prefix_pass:
                    bins = lax.shift_right_logical(b, 16)
                    plsc.addupdate_scatter(hist_v, [bins], ones)
                else:
                    match = lax.shift_right_logical(b, 16) == pvec
                    bins = b & (BINS - 1)
                    plsc.addupdate_scatter(hist_v, [bins], ones, mask=match)

        pltpu.sync_copy(hist_v, out_hbm.at[wid])

    return hist_kernel


_hist_pass1 = _make_hist_kernel(prefix_pass=False)
_hist_pass2 = _make_hist_kernel(prefix_pass=True)


def _pick_bucket(h_key, rank):
    h2d = h_key.reshape(512, 128)
    row_sums = jnp.sum(h2d, axis=1)
    row_cum = jnp.cumsum(row_sums)
    br = jnp.argmax(row_cum >= rank + 1).astype(jnp.int32)
    rank_in_row = rank - (row_cum[br] - row_sums[br])
    row = h2d[br]
    col_cum = jnp.cumsum(row)
    bc = jnp.argmax(col_cum >= rank_in_row + 1).astype(jnp.int32)
    new_rank = rank_in_row - (col_cum[bc] - row[bc])
    return br * 128 + bc, new_rank


def _sigmoid_body(x_ref, t_ref, o_ref):
    t = t_ref[0]
    z = (t - x_ref[...]) * jnp.float32(1.0 / TEMPERATURE)
    o_ref[...] = 1.0 / (1.0 + jnp.exp(z))


def kernel(logits):
    n = logits.shape[-1]
    rank = jnp.int32(n - K_TOP - 1)
    half = BINS // 2

    zeros16 = jnp.zeros((LANES,), jnp.int32)
    h1 = jnp.sum(_hist_pass1(logits, zeros16), axis=0)
    h1_key = jnp.concatenate([h1[half:][::-1], h1[:half]])
    b1, rank = _pick_bucket(h1_key, rank)
    neg = b1 < half
    raw_hi = jnp.where(neg, (BINS - 1) - b1, b1 - half)

    pref = jnp.full((LANES,), raw_hi, jnp.int32)
    h2 = jnp.sum(_hist_pass2(logits, pref), axis=0)
    h2_key = jnp.where(neg, h2[::-1], h2)
    b2, rank = _pick_bucket(h2_key, rank)

    key = (b1 << 16) | b2
    vbits = jnp.where(key < 0, key ^ jnp.int32(MIN_I32), ~key)
    kth_value = lax.bitcast_convert_type(vbits, jnp.float32)

    rows = 4096
    cols = n // rows
    block_rows = 512
    x2 = logits.reshape(rows, cols)
    t1 = kth_value.reshape(1)
    out = pl.pallas_call(
        _sigmoid_body,
        grid=(rows // block_rows,),
        in_specs=[
            pl.BlockSpec((block_rows, cols), lambda i: (i, 0)),
            pl.BlockSpec(memory_space=pltpu.SMEM),
        ],
        out_specs=pl.BlockSpec((block_rows, cols), lambda i: (i, 0)),
        out_shape=jax.ShapeDtypeStruct((rows, cols), jnp.float32),
    )(x2, t1)
    return out.reshape(n)


# speedup vs baseline: 1.0158x; 1.0158x over previous
import functools

import jax
import jax.numpy as jnp
from jax import lax
from jax.experimental import pallas as pl
from jax.experimental.pallas import tpu as pltpu
from jax.experimental.pallas import tpu_sc as plsc

K_TOP = 2048
TEMPERATURE = 0.1

N = 4194304
NUM_CORES = 2
NUM_SUBCORES = 16
NUM_WORKERS = NUM_CORES * NUM_SUBCORES
LANES = 16
SHARD = N // NUM_WORKERS
CHUNK = 16384
NCHUNK = SHARD // CHUNK
BINS = 65536
UNROLL = 8
MIN_I32 = -(2**31)


def _make_hist_kernel(prefix_pass):
    mesh = plsc.VectorSubcoreMesh(core_axis_name="c", subcore_axis_name="s")

    @functools.partial(
        pl.kernel,
        mesh=mesh,
        out_type=jax.ShapeDtypeStruct((NUM_WORKERS * BINS,), jnp.int32),
        compiler_params=pltpu.CompilerParams(needs_layout_passes=False),
        scratch_types=[
            pltpu.VMEM((BINS,), jnp.int32),
            pltpu.VMEM((CHUNK,), jnp.float32),
            pltpu.VMEM((CHUNK,), jnp.float32),
            pltpu.VMEM((LANES,), jnp.int32),
            pltpu.SemaphoreType.DMA,
            pltpu.SemaphoreType.DMA,
        ],
    )
    def hist_kernel(logits_hbm, prefix_hbm, out_hbm, hist_v, buf0, buf1,
                    pref_v, sem0, sem1):
        wid = lax.axis_index("s") * NUM_CORES + lax.axis_index("c")
        base = wid * SHARD
        ones = jnp.ones((LANES,), jnp.int32)
        zeros = jnp.zeros((LANES,), jnp.int32)
        bufs = (buf0, buf1)
        sems = (sem0, sem1)

        pltpu.sync_copy(prefix_hbm, pref_v)
        pvec = pref_v[...]

        copies = [None] * NCHUNK
        copies[0] = pltpu.async_copy(
            logits_hbm.at[pl.ds(base, CHUNK)], buf0, sem0)

        @plsc.parallel_loop(0, BINS // LANES, 1, unroll=UNROLL)
        def _(j):
            hist_v[pl.ds(j * LANES, LANES)] = zeros

        for c in range(NCHUNK):
            if c + 1 < NCHUNK:
                copies[c + 1] = pltpu.async_copy(
                    logits_hbm.at[pl.ds(base + (c + 1) * CHUNK, CHUNK)],
                    bufs[(c + 1) % 2], sems[(c + 1) % 2])
            copies[c].wait()
            buf = bufs[c % 2]

            @plsc.parallel_loop(0, CHUNK // LANES, 1, unroll=UNROLL)
            def _(i):
                x = buf[pl.ds(i * LANES, LANES)]
                b = lax.bitcast_convert_type(x, jnp.int32)
                if not prefix_pass:
                    bins = lax.shift_right_logical(b, 16)
                    plsc.addupdate_scatter(hist_v, [bins], ones)
                else:
                    match = lax.shift_right_logical(b, 16) == pvec
                    bins = b & (BINS - 1)
                    plsc.addupdate_scatter(hist_v, [bins], ones, mask=match)

        pltpu.sync_copy(hist_v, out_hbm.at[pl.ds(wid * BINS, BINS)])

    return hist_kernel


_hist_pass1 = _make_hist_kernel(prefix_pass=False)
_hist_pass2 = _make_hist_kernel(prefix_pass=True)


def _worker_sum(h_flat):
    h = h_flat[0:BINS]
    for w in range(1, NUM_WORKERS):
        h = h + h_flat[w * BINS:(w + 1) * BINS]
    return h


def _pick_bucket(h_key, rank):
    cum = jnp.cumsum(h_key)
    b = jnp.argmax(cum >= rank + 1).astype(jnp.int32)
    new_rank = rank - (cum[b] - h_key[b])
    return b, new_rank


def _sigmoid_body(x_ref, t_ref, o_ref):
    t = t_ref[0]
    z = (t - x_ref[...]) * jnp.float32(1.0 / TEMPERATURE)
    o_ref[...] = 1.0 / (1.0 + jnp.exp(z))


def kernel(logits):
    n = logits.shape[-1]
    rank = jnp.int32(n - K_TOP - 1)
    half = BINS // 2

    zeros16 = jnp.zeros((LANES,), jnp.int32)
    h1 = _worker_sum(_hist_pass1(logits, zeros16))
    h1_key = jnp.concatenate([h1[half:][::-1], h1[:half]])
    b1, rank = _pick_bucket(h1_key, rank)
    neg = b1 < half
    raw_hi = jnp.where(neg, (BINS - 1) - b1, b1 - half)

    pref = jnp.full((LANES,), raw_hi, jnp.int32)
    h2 = _worker_sum(_hist_pass2(logits, pref))
    h2_key = jnp.where(neg, h2[::-1], h2)
    b2, rank = _pick_bucket(h2_key, rank)

    key = (b1 << 16) | b2
    vbits = jnp.where(key < 0, key ^ jnp.int32(MIN_I32), ~key)
    kth_value = lax.bitcast_convert_type(vbits, jnp.float32)

    rows = 4096
    cols = n // rows
    block_rows = 512
    x2 = logits.reshape(rows, cols)
    t1 = kth_value.reshape(1)
    out = pl.pallas_call(
        _sigmoid_body,
        grid=(rows // block_rows,),
        in_specs=[
            pl.BlockSpec((block_rows, cols), lambda i: (i, 0)),
            pl.BlockSpec(memory_space=pltpu.SMEM),
        ],
        out_specs=pl.BlockSpec((block_rows, cols), lambda i: (i, 0)),
        out_shape=jax.ShapeDtypeStruct((rows, cols), jnp.float32),
    )(x2, t1)
    return out.reshape(n)


# speedup vs baseline: 1.2756x; 1.2558x over previous
import functools

import jax
import jax.numpy as jnp
from jax import lax
from jax.experimental import pallas as pl
from jax.experimental.pallas import tpu as pltpu
from jax.experimental.pallas import tpu_sc as plsc

K_TOP = 2048
TEMPERATURE = 0.1

N = 4194304
NUM_CORES = 2
NUM_SUBCORES = 16
NUM_WORKERS = NUM_CORES * NUM_SUBCORES
LANES = 16
SHARD = N // NUM_WORKERS
CHUNK = 16384
NCHUNK = SHARD // CHUNK
BINS = 65536
UNROLL = 8
MIN_I32 = -(2**31)


def _make_hist_kernel(prefix_pass):
    mesh = plsc.VectorSubcoreMesh(core_axis_name="c", subcore_axis_name="s")

    @functools.partial(
        pl.kernel,
        mesh=mesh,
        out_type=jax.ShapeDtypeStruct((NUM_WORKERS * BINS,), jnp.int32),
        compiler_params=pltpu.CompilerParams(needs_layout_passes=False),
        scratch_types=[
            pltpu.VMEM((BINS,), jnp.int32),
            pltpu.VMEM((CHUNK,), jnp.float32),
            pltpu.VMEM((CHUNK,), jnp.float32),
            pltpu.VMEM((LANES,), jnp.int32),
            pltpu.SemaphoreType.DMA,
            pltpu.SemaphoreType.DMA,
        ],
    )
    def hist_kernel(logits_hbm, prefix_hbm, out_hbm, hist_v, buf0, buf1,
                    pref_v, sem0, sem1):
        wid = lax.axis_index("s") * NUM_CORES + lax.axis_index("c")
        base = wid * SHARD
        ones = jnp.ones((LANES,), jnp.int32)
        zeros = jnp.zeros((LANES,), jnp.int32)
        bufs = (buf0, buf1)
        sems = (sem0, sem1)

        pltpu.sync_copy(prefix_hbm, pref_v)
        pvec = pref_v[...]

        copies = [None] * NCHUNK
        copies[0] = pltpu.async_copy(
            logits_hbm.at[pl.ds(base, CHUNK)], buf0, sem0)

        @plsc.parallel_loop(0, BINS // LANES, 1, unroll=UNROLL)
        def _(j):
            hist_v[pl.ds(j * LANES, LANES)] = zeros

        for c in range(NCHUNK):
            if c + 1 < NCHUNK:
                copies[c + 1] = pltpu.async_copy(
                    logits_hbm.at[pl.ds(base + (c + 1) * CHUNK, CHUNK)],
                    bufs[(c + 1) % 2], sems[(c + 1) % 2])
            copies[c].wait()
            buf = bufs[c % 2]

            @plsc.parallel_loop(0, CHUNK // LANES, 1, unroll=UNROLL)
            def _(i):
                x = buf[pl.ds(i * LANES, LANES)]
                b = lax.bitcast_convert_type(x, jnp.int32)
                if not prefix_pass:
                    bins = lax.shift_right_logical(b, 16)
                    plsc.addupdate_scatter(hist_v, [bins], ones)
                else:
                    match = lax.shift_right_logical(b, 16) == pvec
                    bins = b & (BINS - 1)
                    plsc.addupdate_scatter(hist_v, [bins], ones, mask=match)

        pltpu.sync_copy(hist_v, out_hbm.at[pl.ds(wid * BINS, BINS)])

    return hist_kernel


_hist_pass1 = _make_hist_kernel(prefix_pass=False)
_hist_pass2 = _make_hist_kernel(prefix_pass=True)


def _worker_sum(h_flat):
    h = h_flat[0:BINS]
    for w in range(1, NUM_WORKERS):
        h = h + h_flat[w * BINS:(w + 1) * BINS]
    return h


def _pick_bucket(h_key, rank):
    cum = jnp.cumsum(h_key)
    b = jnp.argmax(cum >= rank + 1).astype(jnp.int32)
    new_rank = rank - (cum[b] - h_key[b])
    return b, new_rank


def _sigmoid_body(x_ref, t_ref, o_ref):
    t = t_ref[0]
    z = (t - x_ref[...]) * jnp.float32(1.0 / TEMPERATURE)
    o_ref[...] = 1.0 / (1.0 + jnp.exp(z))


def kernel(logits):
    n = logits.shape[-1]
    rank = jnp.int32(n - K_TOP - 1)
    half = BINS // 2

    zeros16 = jnp.zeros((LANES,), jnp.int32)
    h1 = _worker_sum(_hist_pass1(logits, zeros16))
    h1_key = jnp.concatenate([h1[half:][::-1], h1[:half]])
    b1, rank = _pick_bucket(h1_key, rank)
    neg = b1 < half
    raw_hi = jnp.where(neg, (BINS - 1) - b1, b1 - half)

    pref = jnp.full((LANES,), raw_hi, jnp.int32)
    h2 = _worker_sum(_hist_pass2(logits, pref))
    h2_key = jnp.where(neg, h2[::-1], h2)
    b2, rank = _pick_bucket(h2_key, rank)

    key = (b1 << 16) | b2
    vbits = jnp.where(key < 0, key ^ jnp.int32(MIN_I32), ~key)
    kth_value = lax.bitcast_convert_type(vbits, jnp.float32)

    blk = 524288
    t1 = kth_value.reshape(1)
    out = pl.pallas_call(
        _sigmoid_body,
        grid=(n // blk,),
        in_specs=[
            pl.BlockSpec((blk,), lambda i: (i,)),
            pl.BlockSpec(memory_space=pltpu.SMEM),
        ],
        out_specs=pl.BlockSpec((blk,), lambda i: (i,)),
        out_shape=jax.ShapeDtypeStruct((n,), jnp.float32),
    )(logits, t1)
    return out


# speedup vs baseline: 1.4642x; 1.1478x over previous
import functools

import jax
import jax.numpy as jnp
from jax import lax
from jax.experimental import pallas as pl
from jax.experimental.pallas import tpu as pltpu
from jax.experimental.pallas import tpu_sc as plsc

K_TOP = 2048
TEMPERATURE = 0.1

N = 4194304
NUM_CORES = 2
NUM_SUBCORES = 16
NUM_WORKERS = NUM_CORES * NUM_SUBCORES
LANES = 16
SHARD = N // NUM_WORKERS
CHUNK = 16384
NCHUNK = SHARD // CHUNK
BINS = 65536
ROWS = 512
COLS = 128
UNROLL = 8
MIN_I32 = -(2**31)


def _monotone_key(x_f32):
    b = lax.bitcast_convert_type(x_f32, jnp.int32)
    return b ^ ((b >> 31) | jnp.int32(MIN_I32))


def _make_hist_kernel(prefix_pass):
    mesh = plsc.VectorSubcoreMesh(core_axis_name="c", subcore_axis_name="s")

    @functools.partial(
        pl.kernel,
        mesh=mesh,
        out_type=jax.ShapeDtypeStruct((NUM_WORKERS * BINS,), jnp.int32),
        compiler_params=pltpu.CompilerParams(needs_layout_passes=False),
        scratch_types=[
            pltpu.VMEM((BINS,), jnp.int32),
            pltpu.VMEM((CHUNK,), jnp.float32),
            pltpu.VMEM((CHUNK,), jnp.float32),
            pltpu.VMEM((LANES,), jnp.int32),
            pltpu.SemaphoreType.DMA,
            pltpu.SemaphoreType.DMA,
        ],
    )
    def hist_kernel(logits_hbm, prefix_hbm, out_hbm, hist_v, buf0, buf1,
                    pref_v, sem0, sem1):
        wid = lax.axis_index("s") * NUM_CORES + lax.axis_index("c")
        base = wid * SHARD
        ones = jnp.ones((LANES,), jnp.int32)
        zeros = jnp.zeros((LANES,), jnp.int32)
        bufs = (buf0, buf1)
        sems = (sem0, sem1)

        pltpu.sync_copy(prefix_hbm, pref_v)
        pvec = pref_v[...]

        copies = [None] * NCHUNK
        copies[0] = pltpu.async_copy(
            logits_hbm.at[pl.ds(base, CHUNK)], buf0, sem0)

        @plsc.parallel_loop(0, BINS // LANES, 1, unroll=UNROLL)
        def _(j):
            hist_v[pl.ds(j * LANES, LANES)] = zeros

        for c in range(NCHUNK):
            if c + 1 < NCHUNK:
                copies[c + 1] = pltpu.async_copy(
                    logits_hbm.at[pl.ds(base + (c + 1) * CHUNK, CHUNK)],
                    bufs[(c + 1) % 2], sems[(c + 1) % 2])
            copies[c].wait()
            buf = bufs[c % 2]

            @plsc.parallel_loop(0, CHUNK // LANES, 1, unroll=UNROLL)
            def _(i):
                x = buf[pl.ds(i * LANES, LANES)]
                key = _monotone_key(x)
                if not prefix_pass:
                    bins = lax.shift_right_logical(key, 16)
                    plsc.addupdate_scatter(hist_v, [bins], ones)
                else:
                    match = lax.shift_right_logical(key, 16) == pvec
                    bins = key & (BINS - 1)
                    plsc.addupdate_scatter(hist_v, [bins], ones, mask=match)

        pltpu.sync_copy(hist_v, out_hbm.at[pl.ds(wid * BINS, BINS)])

    return hist_kernel


_hist_pass1 = _make_hist_kernel(prefix_pass=False)
_hist_pass2 = _make_hist_kernel(prefix_pass=True)


def _scan_pick(h_ref, rank):
    acc = h_ref[pl.ds(0, ROWS), :]
    for w in range(1, NUM_WORKERS):
        acc = acc + h_ref[pl.ds(w * ROWS, ROWS), :]
    h2d = acc.astype(jnp.float32)

    rows = jnp.sum(h2d, axis=1, keepdims=True)
    i_r = lax.broadcasted_iota(jnp.int32, (ROWS, ROWS), 0)
    j_r = lax.broadcasted_iota(jnp.int32, (ROWS, ROWS), 1)
    tri_r = (j_r <= i_r).astype(jnp.float32)
    row_cum = jnp.dot(tri_r, rows,
                      preferred_element_type=jnp.float32)

    target = (rank + 1).astype(jnp.float32)
    idx_r = lax.broadcasted_iota(jnp.int32, (ROWS, 1), 0)
    br = jnp.sum((row_cum < target).astype(jnp.int32))
    prev_r = jnp.sum(jnp.where(idx_r == br - 1, row_cum, 0.0))
    rank_in_row = rank - prev_r.astype(jnp.int32)

    idx_r2 = lax.broadcasted_iota(jnp.int32, (ROWS, COLS), 0)
    row = jnp.sum(jnp.where(idx_r2 == br, h2d, 0.0),
                  axis=0, keepdims=True)
    i_c = lax.broadcasted_iota(jnp.int32, (COLS, COLS), 0)
    j_c = lax.broadcasted_iota(jnp.int32, (COLS, COLS), 1)
    tri_ct = (i_c <= j_c).astype(jnp.float32)
    col_cum = jnp.dot(row, tri_ct,
                      preferred_element_type=jnp.float32)

    target2 = (rank_in_row + 1).astype(jnp.float32)
    idx_c = lax.broadcasted_iota(jnp.int32, (1, COLS), 1)
    bc = jnp.sum((col_cum < target2).astype(jnp.int32))
    prev_c = jnp.sum(jnp.where(idx_c == bc - 1, col_cum, 0.0))
    new_rank = rank_in_row - prev_c.astype(jnp.int32)
    return br * COLS + bc, new_rank


def _pick1_body(h_ref, rank_ref, pref_ref, b1_ref, rank_out_ref):
    b, new_rank = _scan_pick(h_ref, rank_ref[0])
    pref_ref[...] = jnp.full((LANES,), b, jnp.int32)
    b1_ref[0] = b
    rank_out_ref[0] = new_rank


def _pick2_body(h_ref, b1_ref, rank_ref, vbits_ref):
    b2, _ = _scan_pick(h_ref, rank_ref[0])
    key = (b1_ref[0] << 16) | b2
    vbits_ref[0] = jnp.where(key < 0, key ^ jnp.int32(MIN_I32), ~key)


def _sigmoid_body(x_ref, t_ref, o_ref):
    t = t_ref[0]
    z = (t - x_ref[...]) * jnp.float32(1.0 / TEMPERATURE)
    o_ref[...] = 1.0 / (1.0 + jnp.exp(z))


_HIST_SPEC = pl.BlockSpec((NUM_WORKERS * ROWS, COLS), lambda: (0, 0))
_SMEM = pl.BlockSpec(memory_space=pltpu.SMEM)


def kernel(logits):
    n = logits.shape[-1]
    rank0 = jnp.full((1,), n - K_TOP - 1, jnp.int32)

    zeros16 = jnp.zeros((LANES,), jnp.int32)
    h1 = _hist_pass1(logits, zeros16).reshape(NUM_WORKERS * ROWS, COLS)
    pref, b1, rank1 = pl.pallas_call(
        _pick1_body,
        in_specs=[_HIST_SPEC, _SMEM],
        out_specs=[pl.BlockSpec((LANES,), lambda: (0,)), _SMEM, _SMEM],
        out_shape=[
            jax.ShapeDtypeStruct((LANES,), jnp.int32),
            jax.ShapeDtypeStruct((1,), jnp.int32),
            jax.ShapeDtypeStruct((1,), jnp.int32),
        ],
    )(h1, rank0)

    h2 = _hist_pass2(logits, pref).reshape(NUM_WORKERS * ROWS, COLS)
    vbits = pl.pallas_call(
        _pick2_body,
        in_specs=[_HIST_SPEC, _SMEM, _SMEM],
        out_specs=_SMEM,
        out_shape=jax.ShapeDtypeStruct((1,), jnp.int32),
    )(h2, b1, rank1)
    kth_value = lax.bitcast_convert_type(vbits, jnp.float32)

    blk = 524288
    out = pl.pallas_call(
        _sigmoid_body,
        grid=(n // blk,),
        in_specs=[
            pl.BlockSpec((blk,), lambda i: (i,)),
            _SMEM,
        ],
        out_specs=pl.BlockSpec((blk,), lambda i: (i,)),
        out_shape=jax.ShapeDtypeStruct((n,), jnp.float32),
    )(logits, kth_value)
    return out
